# Initial kernel scaffold; baseline (speedup 1.0000x reference)
import jax
import jax.numpy as jnp
from jax.experimental import pallas as pl


def kernel(x, gate_w, gate_b, w1, b1, w2, b2):
    raise NotImplementedError("write your pallas kernel here")


# speedup vs baseline: 1.1880x; 1.1880x over previous
import jax
import jax.numpy as jnp
from jax.experimental import pallas as pl
from jax.experimental.pallas import tpu as pltpu

B, S, D, E, F, K = 1, 2048, 768, 8, 2048, 2


def _top2_combine(x, gate_w, gate_b, e):
    logits = jnp.dot(x, gate_w, preferred_element_type=jnp.float32) + gate_b
    iota = jax.lax.broadcasted_iota(jnp.int32, (S, E), 1)
    m0 = jnp.max(logits, axis=1, keepdims=True)
    e0 = jnp.min(jnp.where(logits == m0, iota, E), axis=1, keepdims=True)
    mask0 = iota == e0
    neg = jnp.float32(-1e30)
    l1m = jnp.where(mask0, neg, logits)
    m1 = jnp.max(l1m, axis=1, keepdims=True)
    e1 = jnp.min(jnp.where(l1m == m1, iota, E), axis=1, keepdims=True)
    a = jnp.exp(m1 - m0)
    w0 = 1.0 / (1.0 + a)
    w1 = a / (1.0 + a)
    return jnp.where(e0 == e, w0, 0.0) + jnp.where(e1 == e, w1, 0.0)


def _dense_body(x_ref, gw_ref, gb_ref, w1_ref, b1_ref, w2_ref, b2_ref, out_ref):
    e = pl.program_id(0)
    x = x_ref[...]
    c = _top2_combine(x, gw_ref[...], gb_ref[...], e)
    h = jnp.dot(x, w1_ref[0], preferred_element_type=jnp.float32) + b1_ref[0]
    h = jnp.maximum(h, 0.0)
    o = jnp.dot(h, w2_ref[0], preferred_element_type=jnp.float32) + b2_ref[0]

    @pl.when(e == 0)
    def _():
        out_ref[...] = jnp.zeros_like(out_ref)

    out_ref[...] += c * o


def kernel(x, gate_w, gate_b, w1, b1, w2, b2):
    x2d = x.reshape(S, D)
    gb = gate_b.reshape(1, E)
    b1r = b1.reshape(E, 1, F)
    b2r = b2.reshape(E, 1, D)
    out = pl.pallas_call(
        _dense_body,
        grid=(E,),
        in_specs=[
            pl.BlockSpec((S, D), lambda e: (0, 0)),
            pl.BlockSpec((D, E), lambda e: (0, 0)),
            pl.BlockSpec((1, E), lambda e: (0, 0)),
            pl.BlockSpec((1, D, F), lambda e: (e, 0, 0)),
            pl.BlockSpec((1, 1, F), lambda e: (e, 0, 0)),
            pl.BlockSpec((1, F, D), lambda e: (e, 0, 0)),
            pl.BlockSpec((1, 1, D), lambda e: (e, 0, 0)),
        ],
        out_specs=pl.BlockSpec((S, D), lambda e: (0, 0)),
        out_shape=jax.ShapeDtypeStruct((S, D), jnp.float32),
    )(x2d, gate_w, gb, w1, b1r, w2, b2r)
    return out.reshape(B, S, D)
